# baseline (speedup 1.0000x reference)
import functools

import jax
import jax.numpy as jnp
from jax import lax
from jax.experimental import pallas as pl
from jax.experimental.pallas import tpu as pltpu

K = 20
R = 512
UNROLL = 4


def _stage_body(xall_ref, xblk_ref, w_ref, g_ref, b_ref, *rest):
    if len(rest) == 2:
        a_ref, out_ref = rest
    else:
        (out_ref,) = rest
        a_ref = None

    xall = xall_ref[0]
    xblk = xblk_ref[0]
    n, c = xall.shape
    r = xblk.shape[0]

    n_all = jnp.sum(xall * xall, axis=1)
    n_blk = jnp.sum(xblk * xblk, axis=1)
    inner = lax.dot_general(
        xblk, xall, (((1,), (1,)), ((), ())),
        preferred_element_type=jnp.float32)
    dist = 2.0 * inner - n_blk[:, None] - n_all[None, :]

    w2c = w_ref[...]
    cout = w2c.shape[1]

    xhi = xall.astype(jnp.bfloat16)
    r1 = xall - xhi.astype(jnp.float32)
    xl1 = r1.astype(jnp.bfloat16)
    xl2 = (r1 - xl1.astype(jnp.float32)).astype(jnp.bfloat16)
    xcat = jnp.concatenate([xhi, xl1, xl2], axis=1)

    col = lax.broadcasted_iota(jnp.int32, (r, n), 1)
    gvec = g_ref[...]
    bvec = b_ref[...]
    neg_big = jnp.float32(-3.0e38)

    def body(_, carry):
        d, acc = carry
        m = jnp.max(d, axis=1, keepdims=True)
        idx = jnp.min(jnp.where(d == m, col, n), axis=1,
                      keepdims=True)
        hit = col == idx
        onehot = hit.astype(jnp.bfloat16)
        d = jnp.where(hit, neg_big, d)
        s3 = jnp.dot(onehot, xcat,
                     preferred_element_type=jnp.float32)
        xg = (s3[:, :c] + s3[:, c:2 * c]) + s3[:, 2 * c:]
        feat = jnp.concatenate([xg - xblk, xblk], axis=1)
        y = jnp.dot(feat, w2c, preferred_element_type=jnp.float32)
        mu = jnp.mean(y, axis=1, keepdims=True)
        yc = y - mu
        var = jnp.mean(yc * yc, axis=1, keepdims=True)
        yn = yc * lax.rsqrt(var + 1e-5) * gvec + bvec
        y = jnp.where(yn > 0, yn, 0.2 * yn)
        return d, jnp.maximum(acc, y)

    acc0 = jnp.full((r, cout), neg_big, jnp.float32)

    def body4(_, carry):
        for _ in range(UNROLL):
            carry = body(None, carry)
        return carry

    _, acc = lax.fori_loop(0, K // UNROLL, body4, (dist, acc0))

    if a_ref is not None:
        acc = acc + jnp.dot(xblk, a_ref[...],
                            preferred_element_type=jnp.float32)
    out_ref[0] = acc


def _stage_body_resid(xall_ref, xblk_ref, w_ref, g_ref, b_ref, out_ref):
    _stage_body(xall_ref, xblk_ref, w_ref, g_ref, b_ref, out_ref)
    out_ref[0] = out_ref[0] + xblk_ref[0]


def _edgeconv_stage(xt, w, g, b, a=None, add_identity=False):
    bsz, n, c = xt.shape
    cout = w.shape[0]
    w2c = jnp.transpose(w)
    g2 = g.reshape(1, cout)
    b2 = b.reshape(1, cout)

    in_specs = [
        pl.BlockSpec((1, n, c), lambda i, j: (i, 0, 0)),
        pl.BlockSpec((1, R, c), lambda i, j: (i, j, 0)),
        pl.BlockSpec((2 * c, cout), lambda i, j: (0, 0)),
        pl.BlockSpec((1, cout), lambda i, j: (0, 0)),
        pl.BlockSpec((1, cout), lambda i, j: (0, 0)),
    ]
    args = [xt, xt, w2c, g2, b2]
    if a is not None:
        in_specs.append(pl.BlockSpec((c, cout), lambda i, j: (0, 0)))
        args.append(jnp.transpose(a))
        body = _stage_body
    elif add_identity:
        body = _stage_body_resid
    else:
        body = _stage_body

    return pl.pallas_call(
        body,
        grid=(bsz, n // R),
        in_specs=in_specs,
        out_specs=pl.BlockSpec((1, R, cout), lambda i, j: (i, j, 0)),
        out_shape=jax.ShapeDtypeStruct((bsz, n, cout), jnp.float32),
    )(*args)


@jax.jit
def kernel(x, W1, g1, b1, W2, g2, b2, W3, g3, b3, W4, g4, b4, A1, A2):
    xt = jnp.transpose(x, (0, 2, 1))
    x1 = _edgeconv_stage(xt, W1, g1, b1)
    x2 = _edgeconv_stage(x1, W2, g2, b2, add_identity=True)
    x3 = _edgeconv_stage(x2, W3, g3, b3, a=A1)
    x4 = _edgeconv_stage(x3, W4, g4, b4, a=A2)
    out = jnp.concatenate([x1, x2, x3, x4], axis=2)
    return jnp.transpose(out, (0, 2, 1))


# speedup vs baseline: 1.0350x; 1.0350x over previous
import functools

import jax
import jax.numpy as jnp
from jax import lax
from jax.experimental import pallas as pl
from jax.experimental.pallas import tpu as pltpu

K = 20
R = 512
UNROLL = 10


def _stage_body(xall_ref, xblk_ref, w_ref, g_ref, b_ref, *rest):
    if len(rest) == 2:
        a_ref, out_ref = rest
    else:
        (out_ref,) = rest
        a_ref = None

    xall = xall_ref[0]
    xblk = xblk_ref[0]
    n, c = xall.shape
    r = xblk.shape[0]

    n_all = jnp.sum(xall * xall, axis=1)
    n_blk = jnp.sum(xblk * xblk, axis=1)
    inner = lax.dot_general(
        xblk, xall, (((1,), (1,)), ((), ())),
        preferred_element_type=jnp.float32)
    dist = 2.0 * inner - n_blk[:, None] - n_all[None, :]

    w2c = w_ref[...]
    cout = w2c.shape[1]

    xhi = xall.astype(jnp.bfloat16)
    r1 = xall - xhi.astype(jnp.float32)
    xl1 = r1.astype(jnp.bfloat16)
    xl2 = (r1 - xl1.astype(jnp.float32)).astype(jnp.bfloat16)
    xcat = jnp.concatenate([xhi, xl1, xl2], axis=1)

    col = lax.broadcasted_iota(jnp.int32, (r, n), 1)
    gvec = g_ref[...]
    bvec = b_ref[...]
    neg_big = jnp.float32(-3.0e38)

    def body(_, carry):
        d, acc = carry
        m = jnp.max(d, axis=1, keepdims=True)
        idx = jnp.min(jnp.where(d == m, col, n), axis=1,
                      keepdims=True)
        hit = col == idx
        onehot = hit.astype(jnp.float32)
        d = jnp.where(hit, neg_big, d)
        s3 = jnp.dot(onehot.astype(jnp.bfloat16), xcat,
                     preferred_element_type=jnp.float32)
        xg = (s3[:, :c] + s3[:, c:2 * c]) + s3[:, 2 * c:]
        feat = jnp.concatenate([xg - xblk, xblk], axis=1)
        y = jnp.dot(feat, w2c, preferred_element_type=jnp.float32)
        mu = jnp.mean(y, axis=1, keepdims=True)
        yc = y - mu
        var = jnp.mean(yc * yc, axis=1, keepdims=True)
        yn = yc * lax.rsqrt(var + 1e-5) * gvec + bvec
        y = jnp.where(yn > 0, yn, 0.2 * yn)
        return d, jnp.maximum(acc, y)

    acc0 = jnp.full((r, cout), neg_big, jnp.float32)

    def body4(_, carry):
        for _ in range(UNROLL):
            carry = body(None, carry)
        return carry

    _, acc = lax.fori_loop(0, K // UNROLL, body4, (dist, acc0))

    if a_ref is not None:
        acc = acc + jnp.dot(xblk, a_ref[...],
                            preferred_element_type=jnp.float32)
    out_ref[0] = acc


def _stage_body_resid(xall_ref, xblk_ref, w_ref, g_ref, b_ref, out_ref):
    _stage_body(xall_ref, xblk_ref, w_ref, g_ref, b_ref, out_ref)
    out_ref[0] = out_ref[0] + xblk_ref[0]


def _edgeconv_stage(xt, w, g, b, a=None, add_identity=False):
    bsz, n, c = xt.shape
    cout = w.shape[0]
    w2c = jnp.transpose(w)
    g2 = g.reshape(1, cout)
    b2 = b.reshape(1, cout)

    in_specs = [
        pl.BlockSpec((1, n, c), lambda i, j: (i, 0, 0)),
        pl.BlockSpec((1, R, c), lambda i, j: (i, j, 0)),
        pl.BlockSpec((2 * c, cout), lambda i, j: (0, 0)),
        pl.BlockSpec((1, cout), lambda i, j: (0, 0)),
        pl.BlockSpec((1, cout), lambda i, j: (0, 0)),
    ]
    args = [xt, xt, w2c, g2, b2]
    if a is not None:
        in_specs.append(pl.BlockSpec((c, cout), lambda i, j: (0, 0)))
        args.append(jnp.transpose(a))
        body = _stage_body
    elif add_identity:
        body = _stage_body_resid
    else:
        body = _stage_body

    return pl.pallas_call(
        body,
        grid=(bsz, n // R),
        in_specs=in_specs,
        out_specs=pl.BlockSpec((1, R, cout), lambda i, j: (i, j, 0)),
        out_shape=jax.ShapeDtypeStruct((bsz, n, cout), jnp.float32),
    )(*args)


@jax.jit
def kernel(x, W1, g1, b1, W2, g2, b2, W3, g3, b3, W4, g4, b4, A1, A2):
    xt = jnp.transpose(x, (0, 2, 1))
    x1 = _edgeconv_stage(xt, W1, g1, b1)
    x2 = _edgeconv_stage(x1, W2, g2, b2, add_identity=True)
    x3 = _edgeconv_stage(x2, W3, g3, b3, a=A1)
    x4 = _edgeconv_stage(x3, W4, g4, b4, a=A2)
    out = jnp.concatenate([x1, x2, x3, x4], axis=2)
    return jnp.transpose(out, (0, 2, 1))
